# baseline (speedup 1.0000x reference)
import functools

import jax
import jax.numpy as jnp
from jax import lax
from jax.experimental import pallas as pl
from jax.experimental.pallas import tpu as pltpu
from jax.experimental.pallas import tpu_sc as plsc

_D = 256
_K = 8192
_BM = 1024
_BK = 256
_BETA = 0.25
_SEG_STARTS = (0, 11, 22)
_SEG_ENDS = (10, 21, 31)


def _bf16_round(x):
    return x.astype(jnp.bfloat16).astype(jnp.float32)


def _dist_argmin_body(x_ref, x2_ref, e_ref, e2_ref, idx_ref, loss_ref,
                      seg_min, seg_arg, run_val, fin_arg, fin_min):
    m = pl.program_id(0)
    kb = pl.program_id(1)
    dot = jnp.dot(x_ref[...], e_ref[...], preferred_element_type=jnp.float32)
    d = (x2_ref[...] - 2.0 * dot) + e2_ref[...]
    lmin = jnp.min(d, axis=1)
    col = lax.broadcasted_iota(jnp.int32, d.shape, 1)
    larg = jnp.min(jnp.where(d == lmin[:, None], col, jnp.int32(2**30)),
                   axis=1) + kb * _BK

    is_seg_start = (kb == 0) | (kb == 11) | (kb == 22)

    @pl.when(is_seg_start)
    def _():
        seg_min[...] = lmin
        seg_arg[...] = larg

    @pl.when(jnp.logical_not(is_seg_start))
    def _():
        better = lmin < seg_min[...]
        seg_arg[...] = jnp.where(better, larg, seg_arg[...])
        seg_min[...] = jnp.minimum(seg_min[...], lmin)

    @pl.when(kb == 10)
    def _():
        run_val[...] = _bf16_round(seg_min[...])
        fin_arg[...] = seg_arg[...]
        fin_min[...] = seg_min[...]

    @pl.when((kb == 21) | (kb == 31))
    def _():
        take = seg_min[...] < run_val[...]
        fin_arg[...] = jnp.where(take, seg_arg[...], fin_arg[...])
        fin_min[...] = jnp.where(take, seg_min[...], fin_min[...])
        run_val[...] = jnp.where(take, _bf16_round(seg_min[...]),
                                 run_val[...])

    @pl.when(kb == 31)
    def _():
        idx_ref[...] = fin_arg[...]
        s = jnp.sum(fin_min[...]).reshape(1, 1)

        @pl.when(m == 0)
        def _():
            loss_ref[...] = s

        @pl.when(m > 0)
        def _():
            loss_ref[...] = loss_ref[...] + s


def _dist_argmin(flat, x2, emb, e2):
    m_tot = flat.shape[0]
    grid = (m_tot // _BM, _K // _BK)
    return pl.pallas_call(
        _dist_argmin_body,
        grid=grid,
        in_specs=[
            pl.BlockSpec((_BM, _D), lambda m, k: (m, 0)),
            pl.BlockSpec((_BM, 1), lambda m, k: (m, 0)),
            pl.BlockSpec((_D, _BK), lambda m, k: (0, k)),
            pl.BlockSpec((1, _BK), lambda m, k: (0, k)),
        ],
        out_specs=[
            pl.BlockSpec((_BM,), lambda m, k: (m,)),
            pl.BlockSpec((1, 1), lambda m, k: (0, 0)),
        ],
        out_shape=[
            jax.ShapeDtypeStruct((m_tot,), jnp.int32),
            jax.ShapeDtypeStruct((1, 1), jnp.float32),
        ],
        scratch_shapes=[
            pltpu.VMEM((_BM,), jnp.float32),
            pltpu.VMEM((_BM,), jnp.int32),
            pltpu.VMEM((_BM,), jnp.float32),
            pltpu.VMEM((_BM,), jnp.int32),
            pltpu.VMEM((_BM,), jnp.float32),
        ],
    )(flat, x2, emb, e2)


def _make_gather(b_tot):
    info = plsc.get_sparse_core_info()
    nc, ns = info.num_cores, info.num_subcores
    nw = nc * ns
    b_per_w = b_tot // nw
    ch = 128
    n_ch = b_per_w // ch
    mesh = plsc.VectorSubcoreMesh(core_axis_name="c", subcore_axis_name="s")

    @functools.partial(
        pl.kernel, mesh=mesh,
        out_type=jax.ShapeDtypeStruct((b_tot, _D), jnp.float32),
        scratch_types=[
            pltpu.VMEM((ch,), jnp.int32),
            pltpu.VMEM((ch, _D), jnp.float32),
            pltpu.SemaphoreType.DMA,
        ],
    )
    def gather_k(table_hbm, idx_hbm, out_hbm, idx_v, rows_v, sem):
        wid = lax.axis_index("s") * nc + lax.axis_index("c")
        for c in range(n_ch):
            base = wid * b_per_w + c * ch
            pltpu.sync_copy(idx_hbm.at[pl.ds(base, ch)], idx_v)
            pltpu.async_copy(table_hbm.at[idx_v], rows_v, sem).wait()
            pltpu.sync_copy(rows_v, out_hbm.at[pl.ds(base, ch)])

    return gather_k


def kernel(inputs, embeddings):
    flat = inputs.reshape(-1, _D)
    m_tot = flat.shape[0]
    x2 = jnp.sum(flat ** 2, axis=1, keepdims=True)
    e2 = jnp.sum(embeddings ** 2, axis=0, keepdims=True)
    idx, loss_sum = _dist_argmin(flat, x2, embeddings, e2)
    table = embeddings.T
    quant = _make_gather(m_tot)(table, idx)
    loss = loss_sum[0, 0] * ((1.0 + _BETA) / (m_tot * _D))
    return (quant.reshape(inputs.shape), loss,
            idx.reshape(inputs.shape[:-1]))


# speedup vs baseline: 3.1819x; 3.1819x over previous
import functools

import jax
import jax.numpy as jnp
from jax import lax
from jax.experimental import pallas as pl
from jax.experimental.pallas import tpu as pltpu
from jax.experimental.pallas import tpu_sc as plsc

_D = 256
_K = 8192
_BM = 512
_SEG = 2816
_KPAD = 3 * _SEG
_BETA = 0.25


def _bf16_round(x):
    return x.astype(jnp.bfloat16).astype(jnp.float32)


def _dist_argmin_body(x_ref, x2_ref, e_ref, e2_ref, idx_ref, loss_ref,
                      run_val, fin_arg, fin_min):
    m = pl.program_id(0)
    kb = pl.program_id(1)
    dot = jnp.dot(x_ref[...], e_ref[...], preferred_element_type=jnp.float32)
    d = (x2_ref[...] + dot) + e2_ref[...]
    lmin = jnp.min(d, axis=1)
    col = lax.broadcasted_iota(jnp.int32, d.shape, 1)
    larg = jnp.min(jnp.where(d == lmin[:, None], col, jnp.int32(2**30)),
                   axis=1) + kb * _SEG

    @pl.when(kb == 0)
    def _():
        run_val[...] = _bf16_round(lmin)
        fin_arg[...] = larg
        fin_min[...] = lmin

    @pl.when(kb > 0)
    def _():
        take = lmin < run_val[...]
        fin_arg[...] = jnp.where(take, larg, fin_arg[...])
        fin_min[...] = jnp.where(take, lmin, fin_min[...])
        run_val[...] = jnp.where(take, _bf16_round(lmin), run_val[...])

    @pl.when(kb == 2)
    def _():
        idx_ref[...] = fin_arg[...]
        s = jnp.sum(fin_min[...]).reshape(1, 1)

        @pl.when(m == 0)
        def _():
            loss_ref[...] = s

        @pl.when(m > 0)
        def _():
            loss_ref[...] = loss_ref[...] + s


def _dist_argmin(flat, x2, em2, e2):
    m_tot = flat.shape[0]
    grid = (m_tot // _BM, 3)
    return pl.pallas_call(
        _dist_argmin_body,
        grid=grid,
        in_specs=[
            pl.BlockSpec((_BM, _D), lambda m, k: (m, 0)),
            pl.BlockSpec((_BM, 1), lambda m, k: (m, 0)),
            pl.BlockSpec((_D, _SEG), lambda m, k: (0, k)),
            pl.BlockSpec((1, _SEG), lambda m, k: (0, k)),
        ],
        out_specs=[
            pl.BlockSpec((_BM,), lambda m, k: (m,)),
            pl.BlockSpec((1, 1), lambda m, k: (0, 0)),
        ],
        out_shape=[
            jax.ShapeDtypeStruct((m_tot,), jnp.int32),
            jax.ShapeDtypeStruct((1, 1), jnp.float32),
        ],
        scratch_shapes=[
            pltpu.VMEM((_BM,), jnp.float32),
            pltpu.VMEM((_BM,), jnp.int32),
            pltpu.VMEM((_BM,), jnp.float32),
        ],
    )(flat, x2, em2, e2)


def _make_gather(b_tot):
    info = plsc.get_sparse_core_info()
    nc, ns = info.num_cores, info.num_subcores
    nw = nc * ns
    b_per_w = b_tot // nw
    ch = 128
    n_ch = b_per_w // ch
    mesh = plsc.VectorSubcoreMesh(core_axis_name="c", subcore_axis_name="s")

    @functools.partial(
        pl.kernel, mesh=mesh,
        out_type=jax.ShapeDtypeStruct((b_tot, _D), jnp.float32),
        scratch_types=[
            pltpu.VMEM((ch,), jnp.int32),
            pltpu.VMEM((ch, _D), jnp.float32),
            pltpu.SemaphoreType.DMA,
        ],
    )
    def gather_k(table_hbm, idx_hbm, out_hbm, idx_v, rows_v, sem):
        wid = lax.axis_index("s") * nc + lax.axis_index("c")
        for c in range(n_ch):
            base = wid * b_per_w + c * ch
            pltpu.sync_copy(idx_hbm.at[pl.ds(base, ch)], idx_v)
            pltpu.async_copy(table_hbm.at[idx_v], rows_v, sem).wait()
            pltpu.sync_copy(rows_v, out_hbm.at[pl.ds(base, ch)])

    return gather_k


def kernel(inputs, embeddings):
    flat = inputs.reshape(-1, _D)
    m_tot = flat.shape[0]
    x2 = jnp.sum(flat ** 2, axis=1, keepdims=True)
    e2 = jnp.sum(embeddings ** 2, axis=0, keepdims=True)
    pad = _KPAD - _K
    em2 = jnp.pad(-2.0 * embeddings, ((0, 0), (0, pad)))
    e2p = jnp.pad(e2, ((0, 0), (0, pad)), constant_values=1e30)
    idx, loss_sum = _dist_argmin(flat, x2, em2, e2p)
    table = embeddings.T
    quant = _make_gather(m_tot)(table, idx)
    loss = loss_sum[0, 0] * ((1.0 + _BETA) / (m_tot * _D))
    return (quant.reshape(inputs.shape), loss,
            idx.reshape(inputs.shape[:-1]))


# speedup vs baseline: 3.9484x; 1.2409x over previous
import functools

import jax
import jax.numpy as jnp
from jax import lax
from jax.experimental import pallas as pl
from jax.experimental.pallas import tpu as pltpu
from jax.experimental.pallas import tpu_sc as plsc

_D = 256
_K = 8192
_BM = 512
_SEG = 2816
_KPAD = 3 * _SEG
_BETA = 0.25


def _bf16_round(x):
    return x.astype(jnp.bfloat16).astype(jnp.float32)


def _dist_argmin_body(e_ref, x_ref, x2_ref, e2_ref, idx_ref, loss_ref,
                      run_val, fin_arg, fin_min):
    m = pl.program_id(0)
    kb = pl.program_id(1)
    dot = jnp.dot(e_ref[...], x_ref[...], preferred_element_type=jnp.float32)
    d = (x2_ref[...] + dot) + e2_ref[...]
    lmin = jnp.min(d, axis=0)
    col = lax.broadcasted_iota(jnp.int32, d.shape, 0)
    larg = jnp.min(jnp.where(d == lmin[None, :], col, jnp.int32(2**30)),
                   axis=0) + kb * _SEG

    @pl.when(kb == 0)
    def _():
        run_val[0, :] = _bf16_round(lmin)
        fin_arg[0, :] = larg
        fin_min[0, :] = lmin

    @pl.when(kb > 0)
    def _():
        take = lmin < run_val[0, :]
        fin_arg[0, :] = jnp.where(take, larg, fin_arg[0, :])
        fin_min[0, :] = jnp.where(take, lmin, fin_min[0, :])
        run_val[0, :] = jnp.where(take, _bf16_round(lmin), run_val[0, :])

    @pl.when(kb == 2)
    def _():
        idx_ref[0, 0, :] = fin_arg[0, :]
        s = jnp.sum(fin_min[0, :]).reshape(1, 1)

        @pl.when(m == 0)
        def _():
            loss_ref[...] = s

        @pl.when(m > 0)
        def _():
            loss_ref[...] = loss_ref[...] + s


def _dist_argmin(xt, x2t, em2t, e2t):
    m_tot = xt.shape[1]
    nb = m_tot // _BM
    grid = (nb, 3)
    return pl.pallas_call(
        _dist_argmin_body,
        grid=grid,
        in_specs=[
            pl.BlockSpec((_SEG, _D), lambda m, k: (k, 0)),
            pl.BlockSpec((_D, _BM), lambda m, k: (0, m)),
            pl.BlockSpec((1, _BM), lambda m, k: (0, m)),
            pl.BlockSpec((_SEG, 1), lambda m, k: (k, 0)),
        ],
        out_specs=[
            pl.BlockSpec((1, 1, _BM), lambda m, k: (m, 0, 0)),
            pl.BlockSpec((1, 1), lambda m, k: (0, 0)),
        ],
        out_shape=[
            jax.ShapeDtypeStruct((nb, 1, _BM), jnp.int32),
            jax.ShapeDtypeStruct((1, 1), jnp.float32),
        ],
        scratch_shapes=[
            pltpu.VMEM((1, _BM), jnp.float32),
            pltpu.VMEM((1, _BM), jnp.int32),
            pltpu.VMEM((1, _BM), jnp.float32),
        ],
        compiler_params=pltpu.CompilerParams(
            dimension_semantics=("parallel", "arbitrary")),
    )(em2t, xt, x2t, e2t)


def _make_gather(b_tot):
    info = plsc.get_sparse_core_info()
    nc, ns = info.num_cores, info.num_subcores
    nw = nc * ns
    b_per_w = b_tot // nw
    ch = 128
    n_ch = b_per_w // ch
    mesh = plsc.VectorSubcoreMesh(core_axis_name="c", subcore_axis_name="s")

    @functools.partial(
        pl.kernel, mesh=mesh,
        out_type=jax.ShapeDtypeStruct((b_tot, _D), jnp.float32),
        scratch_types=[
            pltpu.VMEM((ch,), jnp.int32),
            pltpu.VMEM((ch, _D), jnp.float32),
            pltpu.SemaphoreType.DMA,
        ],
    )
    def gather_k(table_hbm, idx_hbm, out_hbm, idx_v, rows_v, sem):
        wid = lax.axis_index("s") * nc + lax.axis_index("c")
        for c in range(n_ch):
            base = wid * b_per_w + c * ch
            pltpu.sync_copy(idx_hbm.at[pl.ds(base, ch)], idx_v)
            pltpu.async_copy(table_hbm.at[idx_v], rows_v, sem).wait()
            pltpu.sync_copy(rows_v, out_hbm.at[pl.ds(base, ch)])

    return gather_k


def kernel(inputs, embeddings):
    flat = inputs.reshape(-1, _D)
    m_tot = flat.shape[0]
    x2 = jnp.sum(flat ** 2, axis=1, keepdims=True)
    e2 = jnp.sum(embeddings ** 2, axis=0, keepdims=True)
    pad = _KPAD - _K
    table = embeddings.T
    em2t = jnp.pad((-2.0 * table).astype(jnp.bfloat16), ((0, pad), (0, 0)))
    e2t = jnp.pad(e2.T, ((0, pad), (0, 0)), constant_values=1e30)
    xt = flat.astype(jnp.bfloat16).T
    idx, loss_sum = _dist_argmin(xt, x2.T, em2t, e2t)
    idx = idx.reshape(-1)
    quant = _make_gather(m_tot)(table, idx)
    loss = loss_sum[0, 0] * ((1.0 + _BETA) / (m_tot * _D))
    return (quant.reshape(inputs.shape), loss,
            idx.reshape(inputs.shape[:-1]))


# speedup vs baseline: 3.9632x; 1.0038x over previous
import functools

import jax
import jax.numpy as jnp
from jax import lax
from jax.experimental import pallas as pl
from jax.experimental.pallas import tpu as pltpu
from jax.experimental.pallas import tpu_sc as plsc

_D = 256
_K = 8192
_BM = 512
_SEG = 2816
_KPAD = 3 * _SEG
_BETA = 0.25


def _bf16_round(x):
    return x.astype(jnp.bfloat16).astype(jnp.float32)


def _dist_argmin_body(e_ref, x_ref, x2_ref, e2_ref, idx_ref, loss_ref,
                      run_val, fin_arg, fin_min):
    m = pl.program_id(0)
    kb = pl.program_id(1)
    dot = lax.dot_general(e_ref[...], x_ref[...], (((1,), (1,)), ((), ())),
                          preferred_element_type=jnp.float32)
    d = (x2_ref[...] + dot) + e2_ref[...]
    lmin = jnp.min(d, axis=0)
    col = lax.broadcasted_iota(jnp.int32, d.shape, 0)
    larg = jnp.min(jnp.where(d == lmin[None, :], col, jnp.int32(2**30)),
                   axis=0) + kb * _SEG

    @pl.when(kb == 0)
    def _():
        run_val[0, :] = _bf16_round(lmin)
        fin_arg[0, :] = larg
        fin_min[0, :] = lmin

    @pl.when(kb > 0)
    def _():
        take = lmin < run_val[0, :]
        fin_arg[0, :] = jnp.where(take, larg, fin_arg[0, :])
        fin_min[0, :] = jnp.where(take, lmin, fin_min[0, :])
        run_val[0, :] = jnp.where(take, _bf16_round(lmin), run_val[0, :])

    @pl.when(kb == 2)
    def _():
        idx_ref[0, 0, :] = fin_arg[0, :]
        s = jnp.sum(fin_min[0, :]).reshape(1, 1)

        @pl.when(m == 0)
        def _():
            loss_ref[...] = s

        @pl.when(m > 0)
        def _():
            loss_ref[...] = loss_ref[...] + s


def _dist_argmin(xb, x2t, em2t, e2t):
    m_tot = xb.shape[0]
    nb = m_tot // _BM
    grid = (nb, 3)
    return pl.pallas_call(
        _dist_argmin_body,
        grid=grid,
        in_specs=[
            pl.BlockSpec((_SEG, _D), lambda m, k: (k, 0)),
            pl.BlockSpec((_BM, _D), lambda m, k: (m, 0)),
            pl.BlockSpec((1, _BM), lambda m, k: (0, m)),
            pl.BlockSpec((_SEG, 1), lambda m, k: (k, 0)),
        ],
        out_specs=[
            pl.BlockSpec((1, 1, _BM), lambda m, k: (m, 0, 0)),
            pl.BlockSpec((1, 1), lambda m, k: (0, 0)),
        ],
        out_shape=[
            jax.ShapeDtypeStruct((nb, 1, _BM), jnp.int32),
            jax.ShapeDtypeStruct((1, 1), jnp.float32),
        ],
        scratch_shapes=[
            pltpu.VMEM((1, _BM), jnp.float32),
            pltpu.VMEM((1, _BM), jnp.int32),
            pltpu.VMEM((1, _BM), jnp.float32),
        ],
        compiler_params=pltpu.CompilerParams(
            dimension_semantics=("parallel", "arbitrary")),
    )(em2t, xb, x2t, e2t)


def _make_gather(b_tot):
    info = plsc.get_sparse_core_info()
    nc, ns = info.num_cores, info.num_subcores
    nw = nc * ns
    b_per_w = b_tot // nw
    ch = 128
    n_ch = b_per_w // ch
    mesh = plsc.VectorSubcoreMesh(core_axis_name="c", subcore_axis_name="s")

    @functools.partial(
        pl.kernel, mesh=mesh,
        out_type=jax.ShapeDtypeStruct((b_tot, _D), jnp.float32),
        scratch_types=[
            pltpu.VMEM((ch,), jnp.int32),
            pltpu.VMEM((ch, _D), jnp.float32),
            pltpu.SemaphoreType.DMA,
        ],
    )
    def gather_k(table_hbm, idx_hbm, out_hbm, idx_v, rows_v, sem):
        wid = lax.axis_index("s") * nc + lax.axis_index("c")
        for c in range(n_ch):
            base = wid * b_per_w + c * ch
            pltpu.sync_copy(idx_hbm.at[pl.ds(base, ch)], idx_v)
            pltpu.async_copy(table_hbm.at[idx_v], rows_v, sem).wait()
            pltpu.sync_copy(rows_v, out_hbm.at[pl.ds(base, ch)])

    return gather_k


def kernel(inputs, embeddings):
    flat = inputs.reshape(-1, _D)
    m_tot = flat.shape[0]
    x2 = jnp.sum(flat ** 2, axis=1, keepdims=True)
    e2 = jnp.sum(embeddings ** 2, axis=0, keepdims=True)
    pad = _KPAD - _K
    table = embeddings.T
    em2t = jnp.pad((-2.0 * table).astype(jnp.bfloat16), ((0, pad), (0, 0)))
    e2t = jnp.pad(e2.T, ((0, pad), (0, 0)), constant_values=1e30)
    xb = flat.astype(jnp.bfloat16)
    idx, loss_sum = _dist_argmin(xb, x2.T, em2t, e2t)
    idx = idx.reshape(-1)
    quant = _make_gather(m_tot)(table, idx)
    loss = loss_sum[0, 0] * ((1.0 + _BETA) / (m_tot * _D))
    return (quant.reshape(inputs.shape), loss,
            idx.reshape(inputs.shape[:-1]))


# speedup vs baseline: 5.0234x; 1.2675x over previous
import functools

import jax
import jax.numpy as jnp
from jax import lax
from jax.experimental import pallas as pl
from jax.experimental.pallas import tpu as pltpu
from jax.experimental.pallas import tpu_sc as plsc

_D = 256
_K = 8192
_BM = 512
_SEG = 2816
_KPAD = 3 * _SEG
_BETA = 0.25


def _bf16_round(x):
    return x.astype(jnp.bfloat16).astype(jnp.float32)


def _dist_argmin_body(e_ref, x_ref, x2_ref, e2_ref, idx_ref, loss_ref,
                      run_val, fin_arg, fin_min):
    m = pl.program_id(0)
    kb = pl.program_id(1)
    dot = lax.dot_general(e_ref[...], x_ref[...], (((1,), (1,)), ((), ())),
                          preferred_element_type=jnp.float32)
    d = (x2_ref[...] + dot) + e2_ref[...]
    lmin = jnp.min(d, axis=0)
    larg = jnp.argmin(d, axis=0).astype(jnp.int32) + kb * _SEG

    @pl.when(kb == 0)
    def _():
        run_val[0, :] = _bf16_round(lmin)
        fin_arg[0, :] = larg
        fin_min[0, :] = lmin

    @pl.when(kb > 0)
    def _():
        take = lmin < run_val[0, :]
        fin_arg[0, :] = jnp.where(take, larg, fin_arg[0, :])
        fin_min[0, :] = jnp.where(take, lmin, fin_min[0, :])
        run_val[0, :] = jnp.where(take, _bf16_round(lmin), run_val[0, :])

    @pl.when(kb == 2)
    def _():
        idx_ref[0, 0, :] = fin_arg[0, :]
        s = jnp.sum(fin_min[0, :]).reshape(1, 1)

        @pl.when(m == 0)
        def _():
            loss_ref[...] = s

        @pl.when(m > 0)
        def _():
            loss_ref[...] = loss_ref[...] + s


def _dist_argmin(xb, x2t, em2t, e2t):
    m_tot = xb.shape[0]
    nb = m_tot // _BM
    grid = (nb, 3)
    return pl.pallas_call(
        _dist_argmin_body,
        grid=grid,
        in_specs=[
            pl.BlockSpec((_SEG, _D), lambda m, k: (k, 0)),
            pl.BlockSpec((_BM, _D), lambda m, k: (m, 0)),
            pl.BlockSpec((1, _BM), lambda m, k: (0, m)),
            pl.BlockSpec((_SEG, 1), lambda m, k: (k, 0)),
        ],
        out_specs=[
            pl.BlockSpec((1, 1, _BM), lambda m, k: (m, 0, 0)),
            pl.BlockSpec((1, 1), lambda m, k: (0, 0)),
        ],
        out_shape=[
            jax.ShapeDtypeStruct((nb, 1, _BM), jnp.int32),
            jax.ShapeDtypeStruct((1, 1), jnp.float32),
        ],
        scratch_shapes=[
            pltpu.VMEM((1, _BM), jnp.float32),
            pltpu.VMEM((1, _BM), jnp.int32),
            pltpu.VMEM((1, _BM), jnp.float32),
        ],
        compiler_params=pltpu.CompilerParams(
            dimension_semantics=("parallel", "arbitrary")),
    )(em2t, xb, x2t, e2t)


def _make_gather(b_tot):
    info = plsc.get_sparse_core_info()
    nc, ns = info.num_cores, info.num_subcores
    nw = nc * ns
    b_per_w = b_tot // nw
    ch = 128
    n_ch = b_per_w // ch
    mesh = plsc.VectorSubcoreMesh(core_axis_name="c", subcore_axis_name="s")

    @functools.partial(
        pl.kernel, mesh=mesh,
        out_type=jax.ShapeDtypeStruct((b_tot, _D), jnp.float32),
        scratch_types=[
            pltpu.VMEM((ch,), jnp.int32),
            pltpu.VMEM((ch, _D), jnp.float32),
            pltpu.SemaphoreType.DMA,
        ],
    )
    def gather_k(table_hbm, idx_hbm, out_hbm, idx_v, rows_v, sem):
        wid = lax.axis_index("s") * nc + lax.axis_index("c")
        for c in range(n_ch):
            base = wid * b_per_w + c * ch
            pltpu.sync_copy(idx_hbm.at[pl.ds(base, ch)], idx_v)
            pltpu.async_copy(table_hbm.at[idx_v], rows_v, sem).wait()
            pltpu.sync_copy(rows_v, out_hbm.at[pl.ds(base, ch)])

    return gather_k


def kernel(inputs, embeddings):
    flat = inputs.reshape(-1, _D)
    m_tot = flat.shape[0]
    x2 = jnp.sum(flat ** 2, axis=1, keepdims=True)
    e2 = jnp.sum(embeddings ** 2, axis=0, keepdims=True)
    pad = _KPAD - _K
    table = embeddings.T
    em2t = jnp.pad((-2.0 * table).astype(jnp.bfloat16), ((0, pad), (0, 0)))
    e2t = jnp.pad(e2.T, ((0, pad), (0, 0)), constant_values=1e30)
    xb = flat.astype(jnp.bfloat16)
    idx, loss_sum = _dist_argmin(xb, x2.T, em2t, e2t)
    idx = idx.reshape(-1)
    quant = _make_gather(m_tot)(table, idx)
    loss = loss_sum[0, 0] * ((1.0 + _BETA) / (m_tot * _D))
    return (quant.reshape(inputs.shape), loss,
            idx.reshape(inputs.shape[:-1]))
